# baseline (speedup 1.0000x reference)
import jax
import jax.numpy as jnp
from jax import lax
from jax.experimental import pallas as pl
from jax.experimental.pallas import tpu as pltpu

_B = 64
_D = 64
_LANES = 128
_W = 64


def _body(idx_ref, *refs):
    xts, ot_ref = refs[:_W], refs[_W]
    i = pl.program_id(0)
    lane_col = lax.broadcasted_iota(jnp.int32, (_LANES, 1), 0)
    out_lane = lax.broadcasted_iota(jnp.int32, (1, _B), 1)
    acc = ot_ref[...]
    for j in range(_W):
        b = i * _W + j
        c = idx_ref[b] % _LANES
        onehot = (lane_col == c).astype(jnp.float32)
        col = jax.lax.dot(
            xts[j][...], onehot, precision=lax.Precision.HIGHEST,
            preferred_element_type=jnp.float32,
        )
        acc = jnp.where(out_lane == b, col, acc)
    ot_ref[...] = acc


def _in_spec(j):
    return pl.BlockSpec(
        (_D, _LANES),
        lambda i, idx_ref, j=j: (0, idx_ref[i * _W + j] // _LANES),
    )


def kernel(x, index):
    xt = x.T
    grid_spec = pltpu.PrefetchScalarGridSpec(
        num_scalar_prefetch=1,
        grid=(_B // _W,),
        in_specs=[_in_spec(j) for j in range(_W)],
        out_specs=pl.BlockSpec((_D, _B), lambda i, idx_ref: (0, 0)),
    )
    out_t = pl.pallas_call(
        _body,
        grid_spec=grid_spec,
        out_shape=jax.ShapeDtypeStruct((_D, _B), jnp.float32),
    )(index, *([xt] * _W))
    return out_t.T


# speedup vs baseline: 1.5966x; 1.5966x over previous
import jax
import jax.numpy as jnp
from jax import lax
from jax.experimental import pallas as pl
from jax.experimental.pallas import tpu as pltpu

_B = 64
_D = 64
_LANES = 128
_W = 64


def _body(idx_ref, *refs):
    xts, ot_ref = refs[:_W], refs[_W]
    lane = lax.broadcasted_iota(jnp.int32, (1, _LANES), 1)
    for j in range(_W):
        c = idx_ref[j] % _LANES
        col = jnp.sum(
            jnp.where(lane == c, xts[j][...], 0.0), axis=1, keepdims=True
        )
        ot_ref[:, j : j + 1] = col


def _in_spec(j):
    return pl.BlockSpec(
        (_D, _LANES),
        lambda i, idx_ref, j=j: (0, idx_ref[i * _W + j] // _LANES),
    )


def kernel(x, index):
    xt = x.T
    grid_spec = pltpu.PrefetchScalarGridSpec(
        num_scalar_prefetch=1,
        grid=(_B // _W,),
        in_specs=[_in_spec(j) for j in range(_W)],
        out_specs=pl.BlockSpec((_D, _B), lambda i, idx_ref: (0, 0)),
    )
    out_t = pl.pallas_call(
        _body,
        grid_spec=grid_spec,
        out_shape=jax.ShapeDtypeStruct((_D, _B), jnp.float32),
    )(index, *([xt] * _W))
    return out_t.T
